# baseline (speedup 1.0000x reference)
import jax
import jax.numpy as jnp
from jax.experimental import pallas as pl
from jax.experimental.pallas import tpu as pltpu

N = 10000
F = 128
H = 64
NC = 2
BR = 400
NBLK = N // BR
BR2 = 1000
NBLK2 = N // BR2
EPS = 1e-5
F8 = jnp.float8_e4m3fn
ASC = 4096.0
XSC = 128.0


def _pass1_kernel(x_ref, g_ref, be_ref, a_ref, x1_ref, a8_ref, xn_ref):
    i = pl.program_id(0)

    @pl.when(i == 0)
    def _():
        xf = x_ref[...]
        m = jnp.mean(xf, axis=0, keepdims=True)
        v = jnp.mean(xf * xf, axis=0, keepdims=True) - m * m
        xn = (xf - m) / jnp.sqrt(v + EPS) * g_ref[...] + be_ref[...]
        xn_ref[...] = xn.astype(F8)

    a8 = (a_ref[...] * ASC).astype(F8)
    a8_ref[...] = a8
    y = jnp.dot(a8, xn_ref[...], preferred_element_type=jnp.float32)
    x1_ref[...] = y * (1.0 / ASC)


def _pass2_kernel(a8_ref, x1_ref, w0_ref, b0_ref, w1_ref, b1_ref,
                  w2_ref, b2_ref, ap_ref, out_ref, xa_ref, xb_ref):
    s = pl.program_id(0)
    i = pl.program_id(1)

    @pl.when(jnp.logical_and(s == 0, i == 0))
    def _():
        xa_ref[...] = (x1_ref[...] * XSC).astype(F8)

    @pl.when(s == 0)
    def _():
        y = jnp.dot(a8_ref[...], xa_ref[...],
                    preferred_element_type=jnp.float32)
        xb_ref[pl.ds(i * BR2, BR2), :] = (y * (1.0 / ASC)).astype(F8)

    @pl.when(s == 1)
    def _():
        y = jnp.dot(a8_ref[...], xb_ref[...],
                    preferred_element_type=jnp.float32)
        y = y * (1.0 / (ASC * XSC))
        ap = ap_ref[...]
        h = jnp.dot(y, w0_ref[...], preferred_element_type=jnp.float32)
        h = h + b0_ref[...]
        h = jnp.where(h >= 0, h, h * ap)
        h = jnp.dot(h, w1_ref[...], preferred_element_type=jnp.float32)
        h = h + b1_ref[...]
        h = jnp.where(h >= 0, h, h * ap)
        o = jnp.dot(h, w2_ref[...], preferred_element_type=jnp.float32)
        o = o + b2_ref[...]
        mx = jnp.max(o, axis=1, keepdims=True)
        lse = mx + jnp.log(jnp.sum(jnp.exp(o - mx), axis=1, keepdims=True))
        out_ref[...] = o - lse


def kernel(x, x_cov, adj, norm_adj, gamma, beta, pW1_0, pb1_0, pWc_0, pbc_0,
           pW1_1, pb1_1, pWc_1, pbc_1, W0, b0, W1m, b1m, W2, b2, a):
    g2 = gamma.reshape(1, F)
    be2 = beta.reshape(1, F)
    b0_2 = b0.reshape(1, H)
    b1_2 = b1m.reshape(1, H)
    b2_2 = b2.reshape(1, NC)
    a2 = jnp.asarray(a, jnp.float32).reshape(1, 1)

    x1, a8 = pl.pallas_call(
        _pass1_kernel,
        grid=(NBLK,),
        in_specs=[
            pl.BlockSpec((N, F), lambda i: (0, 0)),
            pl.BlockSpec((1, F), lambda i: (0, 0)),
            pl.BlockSpec((1, F), lambda i: (0, 0)),
            pl.BlockSpec((BR, N), lambda i: (i, 0)),
        ],
        out_specs=[
            pl.BlockSpec((BR, F), lambda i: (i, 0)),
            pl.BlockSpec((BR, N), lambda i: (i, 0)),
        ],
        out_shape=[
            jax.ShapeDtypeStruct((N, F), jnp.float32),
            jax.ShapeDtypeStruct((N, N), F8),
        ],
        scratch_shapes=[pltpu.VMEM((N, F), F8)],
        compiler_params=pltpu.CompilerParams(
            dimension_semantics=("arbitrary",)),
    )(x, g2, be2, norm_adj)

    out = pl.pallas_call(
        _pass2_kernel,
        grid=(2, NBLK2),
        in_specs=[
            pl.BlockSpec((BR2, N), lambda s, i: (i, 0)),
            pl.BlockSpec((N, F), lambda s, i: (0, 0)),
            pl.BlockSpec((F, H), lambda s, i: (0, 0)),
            pl.BlockSpec((1, H), lambda s, i: (0, 0)),
            pl.BlockSpec((H, H), lambda s, i: (0, 0)),
            pl.BlockSpec((1, H), lambda s, i: (0, 0)),
            pl.BlockSpec((H, NC), lambda s, i: (0, 0)),
            pl.BlockSpec((1, NC), lambda s, i: (0, 0)),
            pl.BlockSpec((1, 1), lambda s, i: (0, 0)),
        ],
        out_specs=pl.BlockSpec((BR2, NC), lambda s, i: (s * i, 0)),
        out_shape=jax.ShapeDtypeStruct((N, NC), jnp.float32),
        scratch_shapes=[
            pltpu.VMEM((N, F), F8),
            pltpu.VMEM((N, F), F8),
        ],
        compiler_params=pltpu.CompilerParams(
            dimension_semantics=("arbitrary", "arbitrary")),
    )(a8, x1, W0, b0_2, W1m, b1_2, W2, b2_2, a2)

    return out
